# Initial kernel scaffold; baseline (speedup 1.0000x reference)
import jax
import jax.numpy as jnp
from jax.experimental import pallas as pl


def kernel(lattice_values, neighbor_idx, weight, bias):
    raise NotImplementedError("write your pallas kernel here")


# speedup vs baseline: 3.5396x; 3.5396x over previous
import functools

import jax
import jax.numpy as jnp
from jax import lax
from jax.experimental import pallas as pl
from jax.experimental.pallas import tpu as pltpu
from jax.experimental.pallas import tpu_sc as plsc

F = 9
D = 128
L = 16
NC = 2
NS = 16
NW = NC * NS
C = 32
G = 96


def _body(n_pad, table, idxf, w_hbm, b_hbm, out_hbm, w_v, b_v, idx_v, rows_v, out_v, sem):
    b_per_w = n_pad // NW
    n_chunks = b_per_w // C
    wid = lax.axis_index("s") * NC + lax.axis_index("c")
    vbase = wid * b_per_w

    pltpu.sync_copy(w_hbm, w_v)
    pltpu.sync_copy(b_hbm, b_v)

    def chunk_body(t, carry):
        v0 = pl.multiple_of(vbase + t * C, C)
        i0 = pl.multiple_of(v0 * F, C * F)
        pltpu.sync_copy(idxf.at[pl.ds(i0, C * F)], idx_v)
        copies = [
            pltpu.async_copy(
                table.at[idx_v.at[pl.ds(j * G, G)]],
                rows_v.at[pl.ds(j * G, G)],
                sem,
            )
            for j in range(C * F // G)
        ]
        for cp in copies:
            cp.wait()

        def c_body(c, carry2):
            r0 = c * F
            for g in range(D // L):
                sl = pl.ds(g * L, L)
                acc = b_v[sl]
                for f in range(F):
                    acc = acc + rows_v[r0 + f, sl] * w_v[f, sl]
                out_v[c, sl] = acc
            return carry2

        lax.fori_loop(0, C, c_body, 0)
        pltpu.sync_copy(out_v, out_hbm.at[pl.ds(v0, C)])
        return carry

    lax.fori_loop(0, n_chunks, chunk_body, 0)


def kernel(lattice_values, neighbor_idx, weight, bias):
    n = lattice_values.shape[0]
    n_pad = -(-n // (NW * C)) * (NW * C)
    idxf = neighbor_idx.astype(jnp.int32).reshape(-1)
    if n_pad != n:
        idxf = jnp.pad(idxf, (0, (n_pad - n) * F))

    mesh = plsc.VectorSubcoreMesh(core_axis_name="c", subcore_axis_name="s")
    run = pl.kernel(
        functools.partial(_body, n_pad),
        out_type=jax.ShapeDtypeStruct((n_pad, D), jnp.float32),
        mesh=mesh,
        scratch_types=[
            pltpu.VMEM((F, D), jnp.float32),
            pltpu.VMEM((D,), jnp.float32),
            pltpu.VMEM((C * F,), jnp.int32),
            pltpu.VMEM((C * F, D), jnp.float32),
            pltpu.VMEM((C, D), jnp.float32),
            pltpu.SemaphoreType.DMA,
        ],
    )
    out = run(lattice_values, idxf, weight, bias)
    return out[:n]
